# baseline (speedup 1.0000x reference)
import functools

import jax
import jax.numpy as jnp
from jax import lax
from jax.experimental import pallas as pl
from jax.experimental.pallas import tpu as pltpu
from jax.experimental.pallas import tpu_sc as plsc

_NC = 2
_NS = 16
_NW = _NC * _NS
_CHUNK = 800


@functools.lru_cache(maxsize=None)
def _make_gather(n_tokens: int, emb: int):
    t_per_w = n_tokens // _NW
    n_chunks = t_per_w // _CHUNK
    assert t_per_w * _NW == n_tokens and n_chunks * _CHUNK == t_per_w
    assert n_chunks % 2 == 0
    half = emb // 2
    mesh = plsc.VectorSubcoreMesh(core_axis_name="c", subcore_axis_name="s")

    @functools.partial(
        pl.kernel,
        out_type=jax.ShapeDtypeStruct((n_tokens, 128), jnp.float32),
        mesh=mesh,
        compiler_params=pltpu.CompilerParams(use_tc_tiling_on_sc=False),
        scratch_types=[
            pltpu.VMEM((2, _CHUNK), jnp.int32),
            pltpu.VMEM((2, _CHUNK), jnp.int32),
            pltpu.VMEM((2, _CHUNK), jnp.int32),
            pltpu.VMEM((2, _CHUNK, half), jnp.float32),
            pltpu.VMEM((2, _CHUNK, half), jnp.float32),
            pltpu.SemaphoreType.DMA,
            pltpu.SemaphoreType.DMA,
            pltpu.SemaphoreType.DMA,
            pltpu.SemaphoreType.DMA,
        ],
    )
    def body(x_hbm, table_hbm, out_hbm, idx_v, ia_v, ib_v, ra_v, rb_v,
             g0, g1, o0, o1):
        wid = lax.axis_index("s") * _NC + lax.axis_index("c")
        gsems = (g0, g1)
        osems = (o0, o1)

        def fire_gather(g, par, sem):
            row = wid * n_chunks + g
            pltpu.sync_copy(x_hbm.at[row], idx_v.at[par])
            for k in range(_CHUNK // 16):
                sl = pl.ds(16 * k, 16)
                two = idx_v[par, sl] * 2
                ia_v[par, sl] = two
                ib_v[par, sl] = two + 1
            pltpu.async_copy(table_hbm.at[ia_v.at[par]], ra_v.at[par], sem)
            pltpu.async_copy(table_hbm.at[ib_v.at[par]], rb_v.at[par], sem)

        def drain_gather(par, sem):
            pltpu.make_async_copy(
                table_hbm.at[ia_v.at[par]], ra_v.at[par], sem
            ).wait()
            pltpu.make_async_copy(
                table_hbm.at[ib_v.at[par]], rb_v.at[par], sem
            ).wait()

        def wb_copies(g, par, sem):
            t0 = (wid * n_chunks + g) * _CHUNK
            return (
                pltpu.make_async_copy(
                    ra_v.at[par],
                    out_hbm.at[pl.ds(t0, _CHUNK), pl.ds(0, half)],
                    sem,
                ),
                pltpu.make_async_copy(
                    rb_v.at[par],
                    out_hbm.at[pl.ds(t0, _CHUNK), pl.ds(half, half)],
                    sem,
                ),
            )

        fire_gather(0, 0, gsems[0])
        fire_gather(1, 1, gsems[1])

        def loop_body(h, carry):
            for par in range(2):
                g = 2 * h + par
                drain_gather(par, gsems[par])
                for c in wb_copies(g, par, osems[par]):
                    c.start()

                @pl.when(g + 2 < n_chunks)
                def _(g=g, par=par):
                    for c in wb_copies(g, par, osems[par]):
                        c.wait()
                    fire_gather(g + 2, par, gsems[par])

            return carry

        lax.fori_loop(0, n_chunks // 2, loop_body, 0)
        for c in wb_copies(n_chunks - 2, 0, osems[0]):
            c.wait()
        for c in wb_copies(n_chunks - 1, 1, osems[1]):
            c.wait()

    return body


def kernel(x, emb_matrix):
    b, l = x.shape
    v, emb = emb_matrix.shape
    n = b * l
    x2d = x.reshape(n // _CHUNK, _CHUNK)
    tbl16 = emb_matrix.reshape(2 * v, emb // 2)
    out128 = _make_gather(n, emb)(x2d, tbl16)
    return out128.reshape(b, l, 128)[:, :, :emb]


# speedup vs baseline: 1.0690x; 1.0690x over previous
import functools

import jax
import jax.numpy as jnp
from jax import lax
from jax.experimental import pallas as pl
from jax.experimental.pallas import tpu as pltpu
from jax.experimental.pallas import tpu_sc as plsc

_NC = 2
_NS = 16
_NW = _NC * _NS
_CHUNK = 1600


@functools.lru_cache(maxsize=None)
def _make_gather(n_tokens: int, emb: int):
    t_per_w = n_tokens // _NW
    n_chunks = t_per_w // _CHUNK
    assert t_per_w * _NW == n_tokens and n_chunks * _CHUNK == t_per_w
    assert n_chunks % 2 == 0
    mesh = plsc.VectorSubcoreMesh(core_axis_name="c", subcore_axis_name="s")

    @functools.partial(
        pl.kernel,
        out_type=jax.ShapeDtypeStruct((n_tokens, 128), jnp.float32),
        mesh=mesh,
        compiler_params=pltpu.CompilerParams(use_tc_tiling_on_sc=False),
        scratch_types=[
            pltpu.VMEM((2, _CHUNK), jnp.int32),
            pltpu.VMEM((2, _CHUNK, emb), jnp.float32),
            pltpu.SemaphoreType.DMA,
            pltpu.SemaphoreType.DMA,
            pltpu.SemaphoreType.DMA,
            pltpu.SemaphoreType.DMA,
        ],
    )
    def body(x_hbm, table_hbm, out_hbm, idx_v, rows_v, g0, g1, o0, o1):
        wid = lax.axis_index("s") * _NC + lax.axis_index("c")
        gsems = (g0, g1)
        osems = (o0, o1)

        def fire_gather(g, par, sem):
            row = wid * n_chunks + g
            pltpu.sync_copy(x_hbm.at[row], idx_v.at[par])
            pltpu.async_copy(
                table_hbm.at[idx_v.at[par]], rows_v.at[par], sem
            )

        def drain_gather(par, sem):
            pltpu.make_async_copy(
                table_hbm.at[idx_v.at[par]], rows_v.at[par], sem
            ).wait()

        def wb_copy(g, par, sem):
            t0 = (wid * n_chunks + g) * _CHUNK
            return pltpu.make_async_copy(
                rows_v.at[par],
                out_hbm.at[pl.ds(t0, _CHUNK), pl.ds(0, emb)],
                sem,
            )

        fire_gather(0, 0, gsems[0])
        fire_gather(1, 1, gsems[1])

        def loop_body(h, carry):
            for par in range(2):
                g = 2 * h + par
                drain_gather(par, gsems[par])
                wb_copy(g, par, osems[par]).start()

                @pl.when(g + 2 < n_chunks)
                def _(g=g, par=par):
                    wb_copy(g, par, osems[par]).wait()
                    fire_gather(g + 2, par, gsems[par])

            return carry

        lax.fori_loop(0, n_chunks // 2, loop_body, 0)
        wb_copy(n_chunks - 2, 0, osems[0]).wait()
        wb_copy(n_chunks - 1, 1, osems[1]).wait()

    return body


def kernel(x, emb_matrix):
    b, l = x.shape
    v, emb = emb_matrix.shape
    n = b * l
    x2d = x.reshape(n // _CHUNK, _CHUNK)
    out128 = _make_gather(n, emb)(x2d, emb_matrix)
    return out128.reshape(b, l, 128)[:, :, :emb]
